# baseline (speedup 1.0000x reference)
import jax
import jax.numpy as jnp
from jax.experimental import pallas as pl

K_NEIGH = 5
ALPHA_SLOPE = 0.2
ROW_BLK = 256
NEG_BIG = -9e15


def _topk_idx(x, mode):
    B, F = x.shape
    kp1 = K_NEIGH + 1

    def kern(xr_ref, xf_ref, idx_ref):
        i = pl.program_id(0)
        xr = xr_ref[...]
        xf = xf_ref[...]
        if mode == "cos":
            nr = jnp.maximum(jnp.sqrt(jnp.sum(xr * xr, axis=1, keepdims=True)), 1e-12)
            nf = jnp.maximum(jnp.sqrt(jnp.sum(xf * xf, axis=1, keepdims=True)), 1e-12)
            score = jax.lax.dot_general(
                xr / nr, xf / nf, (((1,), (1,)), ((), ())),
                preferred_element_type=jnp.float32)
        else:
            dot = jax.lax.dot_general(
                xr, xf, (((1,), (1,)), ((), ())),
                preferred_element_type=jnp.float32)
            sqr = jnp.sum(xr * xr, axis=1, keepdims=True)
            ones = jnp.ones((1, F), jnp.float32)
            sqc = jax.lax.dot_general(
                ones, xf * xf, (((1,), (1,)), ((), ())),
                precision=jax.lax.Precision.HIGHEST,
                preferred_element_type=jnp.float32)
            score = -(sqr + sqc - 2.0 * dot)

        colf = jax.lax.broadcasted_iota(jnp.int32, (ROW_BLK, B), 1).astype(jnp.float32)
        rowid = jax.lax.broadcasted_iota(jnp.int32, (ROW_BLK, 1), 0) + i * ROW_BLK
        for s in range(kp1):
            m = jnp.max(score, axis=1, keepdims=True)
            cand = jnp.where(score == m, colf, jnp.float32(B))
            jf = jnp.min(cand, axis=1, keepdims=True)
            j = jf.astype(jnp.int32)
            valid = j != rowid
            if mode == "cos":
                valid = jnp.logical_and(valid, m > 0.0)
            idx_ref[:, s:s + 1] = jnp.where(valid, j, -1)
            score = jnp.where(colf == jf, -jnp.inf, score)
        idx_ref[:, kp1:kp1 + 2] = jnp.full((ROW_BLK, 2), -1, jnp.int32)

    return pl.pallas_call(
        kern,
        grid=(B // ROW_BLK,),
        in_specs=[
            pl.BlockSpec((ROW_BLK, F), lambda i: (i, 0)),
            pl.BlockSpec((B, F), lambda i: (0, 0)),
        ],
        out_specs=pl.BlockSpec((ROW_BLK, kp1 + 2), lambda i: (i, 0)),
        out_shape=jax.ShapeDtypeStruct((B, kp1 + 2), jnp.int32),
    )(x, x)


def _proj(x, w_all, a1, a2):
    B = x.shape[0]
    P = w_all.shape[1]
    H = a1.shape[1]

    def kern(x_ref, w_ref, a1_ref, a2_ref, wh_ref, s1_ref, s2_ref):
        wh = jnp.dot(x_ref[...], w_ref[...], preferred_element_type=jnp.float32)
        wh_ref[...] = wh
        s1_ref[...] = jnp.dot(wh, a1_ref[...], preferred_element_type=jnp.float32)
        s2_ref[...] = jnp.dot(wh, a2_ref[...], preferred_element_type=jnp.float32)

    return pl.pallas_call(
        kern,
        out_shape=(
            jax.ShapeDtypeStruct((B, P), jnp.float32),
            jax.ShapeDtypeStruct((B, H), jnp.float32),
            jax.ShapeDtypeStruct((B, H), jnp.float32),
        ),
    )(x, w_all, a1, a2)


def _attn_heads(s1b, s2f, whf, maskadd, heads, dh, out_ref):
    maskadd = maskadd.astype(jnp.float32)
    for h in range(heads):
        e = s1b[:, h:h + 1] + s2f[h:h + 1, :]
        e = jnp.maximum(e, ALPHA_SLOPE * e)
        e = e + maskadd
        m = jnp.max(e, axis=1, keepdims=True)
        p = jnp.exp(e - m)
        denom = jnp.sum(p, axis=1, keepdims=True)
        hv = jnp.dot(p, whf[:, h * dh:(h + 1) * dh],
                     preferred_element_type=jnp.float32) / denom
        out_ref[:, h * dh:(h + 1) * dh] = jnp.where(hv > 0, hv, jnp.exp(hv) - 1.0)


def _gat_attn1(s1, s2t, wh_all, idx, idxt, heads, dh):
    B = wh_all.shape[0]
    nslots = idx.shape[1]

    def kern(s1_ref, s2_ref, wh_ref, idx_ref, idxt_ref, out_ref, madd_ref):
        i = pl.program_id(0)
        rowid = jax.lax.broadcasted_iota(jnp.int32, (ROW_BLK, B), 0) + i * ROW_BLK
        colid = jax.lax.broadcasted_iota(jnp.int32, (ROW_BLK, B), 1)
        idx_r = idx_ref[...]
        idxt_f = idxt_ref[...]
        mask = jnp.zeros((ROW_BLK, B), jnp.bool_)
        for s in range(K_NEIGH + 1):
            mask = jnp.logical_or(mask, idx_r[:, s:s + 1] == colid)
            mask = jnp.logical_or(mask, idxt_f[s:s + 1, :] == rowid)
        maskadd = jnp.where(mask, 0.0, jnp.float32(NEG_BIG))
        madd_ref[...] = maskadd.astype(jnp.bfloat16)
        _attn_heads(s1_ref[...], s2_ref[...], wh_ref[...], maskadd,
                    heads, dh, out_ref)

    return pl.pallas_call(
        kern,
        grid=(B // ROW_BLK,),
        in_specs=[
            pl.BlockSpec((ROW_BLK, heads), lambda i: (i, 0)),
            pl.BlockSpec((heads, B), lambda i: (0, 0)),
            pl.BlockSpec((B, heads * dh), lambda i: (0, 0)),
            pl.BlockSpec((ROW_BLK, nslots), lambda i: (i, 0)),
            pl.BlockSpec((nslots, B), lambda i: (0, 0)),
        ],
        out_specs=(
            pl.BlockSpec((ROW_BLK, heads * dh), lambda i: (i, 0)),
            pl.BlockSpec((ROW_BLK, B), lambda i: (i, 0)),
        ),
        out_shape=(
            jax.ShapeDtypeStruct((B, heads * dh), jnp.float32),
            jax.ShapeDtypeStruct((B, B), jnp.bfloat16),
        ),
    )(s1, s2t, wh_all, idx, idxt)


def _gat_attn2(s1, s2t, wh_all, maskadd, heads, dh):
    B = wh_all.shape[0]

    def kern(s1_ref, s2_ref, wh_ref, madd_ref, out_ref):
        _attn_heads(s1_ref[...], s2_ref[...], wh_ref[...], madd_ref[...],
                    heads, dh, out_ref)

    return pl.pallas_call(
        kern,
        grid=(B // ROW_BLK,),
        in_specs=[
            pl.BlockSpec((ROW_BLK, heads), lambda i: (i, 0)),
            pl.BlockSpec((heads, B), lambda i: (0, 0)),
            pl.BlockSpec((B, heads * dh), lambda i: (0, 0)),
            pl.BlockSpec((ROW_BLK, B), lambda i: (i, 0)),
        ],
        out_specs=pl.BlockSpec((ROW_BLK, heads * dh), lambda i: (i, 0)),
        out_shape=jax.ShapeDtypeStruct((B, heads * dh), jnp.float32),
    )(s1, s2t, wh_all, maskadd)


def _mlp(u, f, w0u, w0f, b0, w1, b1, w2, b2):
    B = u.shape[0]

    def kern(u_ref, f_ref, w0u_ref, w0f_ref, b0_ref, w1_ref, b1_ref,
             w2_ref, b2_ref, out_ref):
        h = (jnp.dot(u_ref[...], w0u_ref[...], preferred_element_type=jnp.float32)
             + jnp.dot(f_ref[...], w0f_ref[...], preferred_element_type=jnp.float32)
             + b0_ref[...])
        h = jnp.maximum(h, 0.0)
        h = jnp.dot(h, w1_ref[...], preferred_element_type=jnp.float32) + b1_ref[...]
        h = jnp.maximum(h, 0.0)
        out_ref[...] = (jnp.dot(h, w2_ref[...], preferred_element_type=jnp.float32)
                        + b2_ref[...])

    return pl.pallas_call(
        kern,
        out_shape=jax.ShapeDtypeStruct((B, 1), jnp.float32),
    )(u, f, w0u, w0f, b0, w1, b1, w2, b2)


def _gat_forward(x, idx, w_heads, a_heads, wo, ao):
    H, F, Dh = w_heads.shape
    Do = wo.shape[1]
    idxt = idx.T

    w_all = jnp.transpose(w_heads, (1, 0, 2)).reshape(F, H * Dh)
    rows = jnp.arange(H * Dh)
    a1 = jnp.zeros((H * Dh, H), jnp.float32).at[rows, rows // Dh].set(
        a_heads[:, :Dh, 0].reshape(-1))
    a2 = jnp.zeros((H * Dh, H), jnp.float32).at[rows, rows // Dh].set(
        a_heads[:, Dh:, 0].reshape(-1))
    wh1, s1_1, s2_1 = _proj(x, w_all, a1, a2)
    h1, maskadd = _gat_attn1(s1_1, s2_1.T, wh1, idx, idxt, H, Dh)

    wh2, s1_2, s2_2 = _proj(h1, wo, ao[:Do], ao[Do:])
    return _gat_attn2(s1_2, s2_2.T, wh2, maskadd, 1, Do)


def kernel(user_nodes, food_nodes, uW, ua, uWo, uao, fW, fa, fWo, fao,
           mW0, mb0, mW1, mb1, mW2, mb2):
    idx_u = _topk_idx(user_nodes, "euclid")
    idx_f = _topk_idx(food_nodes, "cos")
    user_emb = _gat_forward(user_nodes, idx_u, uW, ua, uWo, uao)
    food_emb = _gat_forward(food_nodes, idx_f, fW, fa, fWo, fao)
    Do = uWo.shape[1]
    return _mlp(user_emb, food_emb,
                mW0[:Do], mW0[Do:], mb0[None, :],
                mW1, mb1[None, :], mW2, mb2[None, :])


# speedup vs baseline: 1.0046x; 1.0046x over previous
import jax
import jax.numpy as jnp
from jax.experimental import pallas as pl

K_NEIGH = 5
ALPHA_SLOPE = 0.2
ROW_BLK = 256
NEG_BIG = -9e15


def _topk_idx(x, mode):
    B, F = x.shape
    kp1 = K_NEIGH + 1

    def kern(xr_ref, xf_ref, idx_ref):
        i = pl.program_id(0)
        xr = xr_ref[...]
        xf = xf_ref[...]
        if mode == "cos":
            nr = jnp.maximum(jnp.sqrt(jnp.sum(xr * xr, axis=1, keepdims=True)), 1e-12)
            nf = jnp.maximum(jnp.sqrt(jnp.sum(xf * xf, axis=1, keepdims=True)), 1e-12)
            score = jax.lax.dot_general(
                xr / nr, xf / nf, (((1,), (1,)), ((), ())),
                preferred_element_type=jnp.float32)
        else:
            dot = jax.lax.dot_general(
                xr, xf, (((1,), (1,)), ((), ())),
                preferred_element_type=jnp.float32)
            sqr = jnp.sum(xr * xr, axis=1, keepdims=True)
            ones = jnp.ones((1, F), jnp.float32)
            sqc = jax.lax.dot_general(
                ones, xf * xf, (((1,), (1,)), ((), ())),
                precision=jax.lax.Precision.HIGHEST,
                preferred_element_type=jnp.float32)
            score = -(sqr + sqc - 2.0 * dot)

        colf = jax.lax.broadcasted_iota(jnp.int32, (ROW_BLK, B), 1).astype(jnp.float32)
        rowid = jax.lax.broadcasted_iota(jnp.int32, (ROW_BLK, 1), 0) + i * ROW_BLK
        for s in range(kp1):
            m = jnp.max(score, axis=1, keepdims=True)
            cand = jnp.where(score == m, colf, jnp.float32(B))
            jf = jnp.min(cand, axis=1, keepdims=True)
            j = jf.astype(jnp.int32)
            valid = j != rowid
            if mode == "cos":
                valid = jnp.logical_and(valid, m > 0.0)
            idx_ref[:, s:s + 1] = jnp.where(valid, j, -1)
            score = jnp.where(colf == jf, -jnp.inf, score)
        idx_ref[:, kp1:kp1 + 2] = jnp.full((ROW_BLK, 2), -1, jnp.int32)

    return pl.pallas_call(
        kern,
        grid=(B // ROW_BLK,),
        in_specs=[
            pl.BlockSpec((ROW_BLK, F), lambda i: (i, 0)),
            pl.BlockSpec((B, F), lambda i: (0, 0)),
        ],
        out_specs=pl.BlockSpec((ROW_BLK, kp1 + 2), lambda i: (i, 0)),
        out_shape=jax.ShapeDtypeStruct((B, kp1 + 2), jnp.int32),
    )(x, x)


def _proj(x, w_all, a1, a2):
    B = x.shape[0]
    P = w_all.shape[1]
    H = a1.shape[1]

    def kern(x_ref, w_ref, a1_ref, a2_ref, wh_ref, s1_ref, s2_ref):
        wh = jnp.dot(x_ref[...], w_ref[...], preferred_element_type=jnp.float32)
        wh_ref[...] = wh
        s1_ref[...] = jnp.dot(wh, a1_ref[...], preferred_element_type=jnp.float32)
        s2_ref[...] = jnp.dot(wh, a2_ref[...], preferred_element_type=jnp.float32)

    return pl.pallas_call(
        kern,
        out_shape=(
            jax.ShapeDtypeStruct((B, P), jnp.float32),
            jax.ShapeDtypeStruct((B, H), jnp.float32),
            jax.ShapeDtypeStruct((B, H), jnp.float32),
        ),
    )(x, w_all, a1, a2)


def _attn_heads(s1b, s2f, whf, maskadd, heads, dh, out_ref):
    for h in range(heads):
        e = s1b[:, h:h + 1] + s2f[h:h + 1, :]
        e = jnp.maximum(e, ALPHA_SLOPE * e)
        e = e + maskadd
        m = jnp.max(e, axis=1, keepdims=True)
        p = jnp.exp(e - m)
        denom = jnp.sum(p, axis=1, keepdims=True)
        hv = jnp.dot(p, whf[:, h * dh:(h + 1) * dh],
                     preferred_element_type=jnp.float32) / denom
        out_ref[:, h * dh:(h + 1) * dh] = jnp.where(hv > 0, hv, jnp.exp(hv) - 1.0)


def _gat_attn1(s1, s2t, wh_all, idx, idxt, heads, dh):
    B = wh_all.shape[0]
    nslots = idx.shape[1]

    def kern(s1_ref, s2_ref, wh_ref, idx_ref, idxt_ref, out_ref, madd_ref):
        i = pl.program_id(0)
        rowid = jax.lax.broadcasted_iota(jnp.int32, (ROW_BLK, B), 0) + i * ROW_BLK
        colid = jax.lax.broadcasted_iota(jnp.int32, (ROW_BLK, B), 1)
        idx_r = idx_ref[...]
        idxt_f = idxt_ref[...]
        mask = jnp.zeros((ROW_BLK, B), jnp.bool_)
        for s in range(K_NEIGH + 1):
            mask = jnp.logical_or(mask, idx_r[:, s:s + 1] == colid)
            mask = jnp.logical_or(mask, idxt_f[s:s + 1, :] == rowid)
        maskadd = jnp.where(mask, 0.0, jnp.float32(NEG_BIG))
        madd_ref[...] = maskadd
        _attn_heads(s1_ref[...], s2_ref[...], wh_ref[...], maskadd,
                    heads, dh, out_ref)

    return pl.pallas_call(
        kern,
        grid=(B // ROW_BLK,),
        in_specs=[
            pl.BlockSpec((ROW_BLK, heads), lambda i: (i, 0)),
            pl.BlockSpec((heads, B), lambda i: (0, 0)),
            pl.BlockSpec((B, heads * dh), lambda i: (0, 0)),
            pl.BlockSpec((ROW_BLK, nslots), lambda i: (i, 0)),
            pl.BlockSpec((nslots, B), lambda i: (0, 0)),
        ],
        out_specs=(
            pl.BlockSpec((ROW_BLK, heads * dh), lambda i: (i, 0)),
            pl.BlockSpec((ROW_BLK, B), lambda i: (i, 0)),
        ),
        out_shape=(
            jax.ShapeDtypeStruct((B, heads * dh), jnp.float32),
            jax.ShapeDtypeStruct((B, B), jnp.float32),
        ),
    )(s1, s2t, wh_all, idx, idxt)


def _gat_attn2(s1, s2t, wh_all, maskadd, heads, dh):
    B = wh_all.shape[0]

    def kern(s1_ref, s2_ref, wh_ref, madd_ref, out_ref):
        _attn_heads(s1_ref[...], s2_ref[...], wh_ref[...], madd_ref[...],
                    heads, dh, out_ref)

    return pl.pallas_call(
        kern,
        grid=(B // ROW_BLK,),
        in_specs=[
            pl.BlockSpec((ROW_BLK, heads), lambda i: (i, 0)),
            pl.BlockSpec((heads, B), lambda i: (0, 0)),
            pl.BlockSpec((B, heads * dh), lambda i: (0, 0)),
            pl.BlockSpec((ROW_BLK, B), lambda i: (i, 0)),
        ],
        out_specs=pl.BlockSpec((ROW_BLK, heads * dh), lambda i: (i, 0)),
        out_shape=jax.ShapeDtypeStruct((B, heads * dh), jnp.float32),
    )(s1, s2t, wh_all, maskadd)


def _mlp(u, f, w0u, w0f, b0, w1, b1, w2, b2):
    B = u.shape[0]

    def kern(u_ref, f_ref, w0u_ref, w0f_ref, b0_ref, w1_ref, b1_ref,
             w2_ref, b2_ref, out_ref):
        h = (jnp.dot(u_ref[...], w0u_ref[...], preferred_element_type=jnp.float32)
             + jnp.dot(f_ref[...], w0f_ref[...], preferred_element_type=jnp.float32)
             + b0_ref[...])
        h = jnp.maximum(h, 0.0)
        h = jnp.dot(h, w1_ref[...], preferred_element_type=jnp.float32) + b1_ref[...]
        h = jnp.maximum(h, 0.0)
        out_ref[...] = (jnp.dot(h, w2_ref[...], preferred_element_type=jnp.float32)
                        + b2_ref[...])

    return pl.pallas_call(
        kern,
        out_shape=jax.ShapeDtypeStruct((B, 1), jnp.float32),
    )(u, f, w0u, w0f, b0, w1, b1, w2, b2)


def _gat_forward(x, idx, w_heads, a_heads, wo, ao):
    H, F, Dh = w_heads.shape
    Do = wo.shape[1]
    idxt = idx.T

    w_all = jnp.transpose(w_heads, (1, 0, 2)).reshape(F, H * Dh)
    rows = jnp.arange(H * Dh)
    a1 = jnp.zeros((H * Dh, H), jnp.float32).at[rows, rows // Dh].set(
        a_heads[:, :Dh, 0].reshape(-1))
    a2 = jnp.zeros((H * Dh, H), jnp.float32).at[rows, rows // Dh].set(
        a_heads[:, Dh:, 0].reshape(-1))
    wh1, s1_1, s2_1 = _proj(x, w_all, a1, a2)
    h1, maskadd = _gat_attn1(s1_1, s2_1.T, wh1, idx, idxt, H, Dh)

    wh2, s1_2, s2_2 = _proj(h1, wo, ao[:Do], ao[Do:])
    return _gat_attn2(s1_2, s2_2.T, wh2, maskadd, 1, Do)


def kernel(user_nodes, food_nodes, uW, ua, uWo, uao, fW, fa, fWo, fao,
           mW0, mb0, mW1, mb1, mW2, mb2):
    idx_u = _topk_idx(user_nodes, "euclid")
    idx_f = _topk_idx(food_nodes, "cos")
    user_emb = _gat_forward(user_nodes, idx_u, uW, ua, uWo, uao)
    food_emb = _gat_forward(food_nodes, idx_f, fW, fa, fWo, fao)
    Do = uWo.shape[1]
    return _mlp(user_emb, food_emb,
                mW0[:Do], mW0[Do:], mb0[None, :],
                mW1, mb1[None, :], mW2, mb2[None, :])
